# baseline (speedup 1.0000x reference)
import dataclasses
import functools

import jax
import jax.numpy as jnp
from jax import lax
from jax.experimental import pallas as pl
from jax.experimental.pallas import tpu as pltpu
from jax.experimental.pallas import tpu_sc as plsc

GX, GY = 432, 496
NCELL = GX * GY
MAXV, MAXP, C = 20000, 32, 4
N = 200000
NT = 16
CPT = NCELL // NT
PPT = N // NT
PPT_PAD = 14336
NVEC_A = 782

REGDUMP = 200128
REG = 202240

VOX_WORDS = 2560512
VOXDUMPW = 2560480
COOR_WORDS = 60160
COORDUMPW = 60040
NPV_WORDS = 20096
NPVDUMP = 20088

VS0, VS1, VS2 = 0.16, 0.16, 4.0
PR0, PR1, PR2 = 0.0, -39.68, -3.0

_MESH = plsc.VectorSubcoreMesh(core_axis_name="c", subcore_axis_name="s",
                               num_cores=1)
_CP = pltpu.CompilerParams()
if "needs_layout_passes" in pltpu.CompilerParams.__dataclass_fields__:
    _CP = dataclasses.replace(_CP, needs_layout_passes=False)


def _floor_div(q):
    ti = q.astype(jnp.int32)
    return ti - (ti.astype(jnp.float32) > q).astype(jnp.int32)


@functools.partial(
    pl.kernel,
    out_type=[
        jax.ShapeDtypeStruct((VOX_WORDS,), jnp.float32),
        jax.ShapeDtypeStruct((COOR_WORDS,), jnp.int32),
        jax.ShapeDtypeStruct((NPV_WORDS,), jnp.int32),
        jax.ShapeDtypeStruct((16,), jnp.int32),
    ],
    mesh=_MESH,
    compiler_params=_CP,
    scratch_types=[
        pltpu.HBM((REG,), jnp.int32),
        pltpu.HBM((REG,), jnp.int32),
        pltpu.HBM((256,), jnp.int32),
        pltpu.HBM((256,), jnp.int32),
        pltpu.VMEM((51200,), jnp.float32),
        pltpu.VMEM((PPT_PAD,), jnp.int32),
        pltpu.VMEM((PPT_PAD,), jnp.int32),
        pltpu.VMEM((32,), jnp.int32),
        pltpu.VMEM((256,), jnp.int32),
        pltpu.VMEM((32,), jnp.int32),
        pltpu.VMEM((CPT,), jnp.int32),
        pltpu.VMEM((2048,), jnp.int32),
        pltpu.VMEM((2048,), jnp.int32),
        *[pltpu.VMEM((2048,), jnp.int32) for _ in range(7)],
        pltpu.VMEM((2048,), jnp.float32),
        pltpu.VMEM((2048,), jnp.int32),
        pltpu.VMEM((2048,), jnp.int32),
        pltpu.VMEM((512,), jnp.int32),
        pltpu.VMEM((1536,), jnp.int32),
        pltpu.VMEM((1536,), jnp.int32),
        pltpu.VMEM((512,), jnp.int32),
        pltpu.VMEM((512,), jnp.int32),
        pltpu.VMEM((16,), jnp.int32),
        pltpu.SemaphoreType.DMA,
    ],
)
def _vox_kernel(pts_hbm, vox_out, coor_out, npv_out, vnum_out,
                rc_hbm, rs_hbm, cmat_hbm, occ_hbm,
                pts, lin_all, gidx_all, ho, cm, base_tbl, hcell,
                vstage, istage, i2a, i2b, i2c, i2d, i2e, i2f, i2g, zf, zi, widx, slbuf,
                cw, cwi, npvv, npvi, b16, sem):
    istage7 = [i2a, i2b, i2c, i2d, i2e, i2f, i2g]
    t = lax.axis_index("s")
    lane = lax.iota(jnp.int32, 16)
    ones = jnp.ones((16,), jnp.int32)
    zeros16 = jnp.zeros((16,), jnp.int32)

    pltpu.sync_copy(pts_hbm.at[pl.ds(pl.multiple_of(t * (PPT * C), 8),
                                     PPT * C)],
                    pts.at[pl.ds(0, PPT * C)])
    ho[pl.ds(0, 16)] = zeros16
    ho[pl.ds(16, 16)] = zeros16

    def a_body(i, _):
        base = i * 16
        lanes = base + lane
        pm = lanes < PPT
        x = plsc.load_gather(pts, [lanes * 4])
        y = plsc.load_gather(pts, [lanes * 4 + 1])
        z = plsc.load_gather(pts, [lanes * 4 + 2])
        cx = _floor_div((x - PR0) / jnp.float32(VS0))
        cy = _floor_div((y - PR1) / jnp.float32(VS1))
        cz = _floor_div((z - PR2) / jnp.float32(VS2))
        valid = ((cx >= 0) & (cx < GX) & (cy >= 0) & (cy < GY)
                 & (cz == 0) & pm)
        lin = jnp.where(valid, cy * GX + cx, NCELL)
        lin_all[pl.ds(base, 16)] = lin
        owner = lin // CPT
        prior, _unused = plsc.scan_count(owner, mask=valid)
        hbase = plsc.load_gather(ho, [owner], mask=valid)
        loff = hbase + prior - 1
        plsc.addupdate_scatter(ho, [owner], ones, mask=valid)
        gidx_all[pl.ds(base, 16)] = loff
        return 0

    lax.fori_loop(0, NVEC_A, a_body, 0)

    b16[...] = ho[pl.ds(0, 16)]
    pltpu.sync_copy(b16, cmat_hbm.at[pl.ds(pl.multiple_of(t * 16, 8), 16)])
    plsc.subcore_barrier()
    pltpu.sync_copy(cmat_hbm, cm)

    accs = jnp.zeros((16,), jnp.int32)
    tot = jnp.zeros((16,), jnp.int32)
    for tp in range(NT):
        row = cm[pl.ds(tp * 16, 16)]
        accs = accs + row * (jnp.int32(tp) < t).astype(jnp.int32)
        tot = tot + row
    tot8 = (tot + 7) & ~7
    regs = plsc.cumsum(tot8) - tot8
    base_tbl[pl.ds(0, 16)] = regs + accs
    base_tbl[pl.ds(16, 16)] = zeros16
    my_r = pl.multiple_of(jnp.sum(jnp.where(lane == t, regs, 0)), 8)
    n_mine = jnp.sum(jnp.where(lane == t, tot, 0))

    copies = []
    for c0 in range(7):
        ibuf = istage7[c0]

        def c_pre(j, _, c0=c0, ibuf=ibuf):
            base = c0 * 2048 + j * 16
            lanes = base + lane
            pm = lanes < PPT
            linv = lin_all[pl.ds(base, 16)]
            ok = pm & (linv < NCELL)
            owner = jnp.where(ok, linv // CPT, 0)
            slot = jnp.where(ok,
                             plsc.load_gather(base_tbl, [owner])
                             + gidx_all[pl.ds(base, 16)],
                             REGDUMP)
            gidx_all[pl.ds(base, 16)] = slot
            ibuf[pl.ds(j * 16, 16)] = slot
            return 0

        lax.fori_loop(0, 128, c_pre, 0)
        copies.append(pltpu.async_copy(
            lin_all.at[pl.ds(c0 * 2048, 2048)],
            rc_hbm.at[ibuf], sem))
    for cp in copies:
        cp.wait()
    return

    def z_body(j, _):
        zf[pl.ds(j * 16, 16)] = jnp.zeros((16,), jnp.float32)
        zi[pl.ds(j * 16, 16)] = zeros16
        return 0

    lax.fori_loop(0, 128, z_body, 0)
    vz = pl.multiple_of(t * 160032, 8)
    for k in range(78):
        pltpu.sync_copy(zf, vox_out.at[pl.ds(vz + k * 2048, 2048)])
    pltpu.sync_copy(zf.at[pl.ds(0, 288)],
                    vox_out.at[pl.ds(vz + 78 * 2048, 288)])
    pltpu.sync_copy(zi.at[pl.ds(0, 1256)],
                    npv_out.at[pl.ds(pl.multiple_of(t * 1256, 8), 1256)])

    def zneg_body(j, _):
        zi[pl.ds(j * 16, 16)] = jnp.full((16,), -1, jnp.int32)
        return 0

    lax.fori_loop(0, 128, zneg_body, 0)
    cz0 = pl.multiple_of(t * 3760, 8)
    pltpu.sync_copy(zi, coor_out.at[pl.ds(cz0, 2048)])
    pltpu.sync_copy(zi.at[pl.ds(0, 1712)],
                    coor_out.at[pl.ds(cz0 + 2048, 1712)])

    plsc.subcore_barrier()

    def hz_body(j, _):
        hcell[pl.ds(j * 16, 16)] = zeros16
        return 0

    lax.fori_loop(0, CPT // 16, hz_body, 0)
    nchunks = (n_mine + 2047) // 2048

    def d1_chunk(c0, _):
        pltpu.sync_copy(rc_hbm.at[pl.ds(pl.multiple_of(my_r + c0 * 2048, 8), 2048)], vstage)

        def d1_body(j, _):
            li = c0 * 2048 + j * 16 + lane
            m = li < n_mine
            cell = jnp.where(m, vstage[pl.ds(j * 16, 16)] - t * CPT, 0)
            plsc.addupdate_scatter(hcell, [cell], ones, mask=m)
            return 0

        lax.fori_loop(0, 128, d1_body, 0)
        return 0

    lax.fori_loop(0, nchunks, d1_chunk, 0)

    def d2_body(j, carry):
        h = hcell[pl.ds(j * 16, 16)]
        occ = (h > 0).astype(jnp.int32)
        excl = plsc.cumsum(occ) - occ + carry
        hcell[pl.ds(j * 16, 16)] = excl << 16
        return carry + jnp.sum(occ)

    occ_t = lax.fori_loop(0, CPT // 16, d2_body, jnp.int32(0))
    b16[...] = jnp.full((16,), occ_t, jnp.int32)
    pltpu.sync_copy(b16, occ_hbm.at[pl.ds(pl.multiple_of(t * 16, 8), 16)])
    plsc.subcore_barrier()
    pltpu.sync_copy(occ_hbm, cm)
    occv = plsc.load_gather(cm, [lane * 16])
    rb = jnp.sum(jnp.where(lane < t, occv, 0))
    total_occ = jnp.sum(occv)

    def d3_chunk(c0, _):
        pltpu.sync_copy(rc_hbm.at[pl.ds(pl.multiple_of(my_r + c0 * 2048, 8), 2048)], vstage)

        def d3_body(j, _):
            li = c0 * 2048 + j * 16 + lane
            m = li < n_mine
            cell = jnp.where(m, vstage[pl.ds(j * 16, 16)] - t * CPT, 0)
            h = plsc.load_gather(hcell, [cell], mask=m)
            prior, _u = plsc.scan_count(cell, mask=m)
            pos = (h & 0xFFFF) + prior - 1
            rank = rb + (h >> 16)
            plsc.addupdate_scatter(hcell, [cell], ones, mask=m)
            keep = m & (pos < MAXP) & (rank < MAXV)
            slot = jnp.where(keep, rank * MAXP + pos, 640000)
            vstage[pl.ds(j * 16, 16)] = slot
            istage[pl.ds(j * 16, 16)] = jnp.where(m, my_r + li, REGDUMP)
            return 0

        lax.fori_loop(0, 128, d3_body, 0)
        pltpu.async_copy(vstage, rs_hbm.at[istage], sem).wait()
        return 0

    lax.fori_loop(0, nchunks, d3_chunk, 0)

    def co_chunk(c0, _):
        def co_body(j, _):
            cl = c0 * 512 + j * 16 + lane
            inr = cl < CPT
            clc = jnp.where(inr, cl, 0)
            h = plsc.load_gather(hcell, [clc])
            cnt = h & 0xFFFF
            r = rb + (h >> 16)
            ok = inr & (cnt > 0) & (r < MAXV)
            g = t * CPT + clc
            yv = g // GX
            xv = g - yv * GX
            p = (j * 16 + lane) * 3
            plsc.store_scatter(cw, [p], zeros16)
            plsc.store_scatter(cw, [p + 1], yv)
            plsc.store_scatter(cw, [p + 2], xv)
            wbase = jnp.where(ok, r * 3, COORDUMPW)
            plsc.store_scatter(cwi, [p], wbase)
            plsc.store_scatter(cwi, [p + 1], wbase + 1)
            plsc.store_scatter(cwi, [p + 2], wbase + 2)
            q = j * 16 + lane
            plsc.store_scatter(npvv, [q], jnp.minimum(cnt, MAXP))
            plsc.store_scatter(npvi, [q], jnp.where(ok, r, NPVDUMP))
            return 0

        lax.fori_loop(0, 32, co_body, 0)
        pltpu.async_copy(cw, coor_out.at[cwi], sem).wait()
        pltpu.async_copy(npvv, npv_out.at[npvi], sem).wait()
        return 0

    lax.fori_loop(0, 27, co_chunk, 0)

    @pl.when(t == 0)
    def _():
        b16[...] = jnp.full((16,), jnp.minimum(total_occ, MAXV), jnp.int32)
        pltpu.sync_copy(b16, vnum_out)

    plsc.subcore_barrier()

    def e_chunk(c0, _):
        pltpu.async_copy(rs_hbm.at[gidx_all.at[pl.ds(c0 * 512, 512)]],
                         slbuf, sem).wait()

        def e_body(j, _):
            base = c0 * 512 + j * 16
            lanes = base + lane
            okp = (lanes < PPT) & (lin_all[pl.ds(base, 16)] < NCELL)
            sl = slbuf[pl.ds(j * 16, 16)]
            wv = jnp.where(okp, sl * 4, VOXDUMPW)
            p = (j * 16 + lane) * 4
            plsc.store_scatter(widx, [p], wv)
            plsc.store_scatter(widx, [p + 1], wv + 1)
            plsc.store_scatter(widx, [p + 2], wv + 2)
            plsc.store_scatter(widx, [p + 3], wv + 3)
            return 0

        lax.fori_loop(0, 32, e_body, 0)
        pltpu.async_copy(pts.at[pl.ds(c0 * 2048, 2048)],
                         vox_out.at[widx], sem).wait()
        return 0

    lax.fori_loop(0, 25, e_chunk, 0)


def kernel(points):
    pts_flat = points.reshape(-1)
    vox, coor, npv, vnum = _vox_kernel(pts_flat)
    voxels = vox[: MAXV * MAXP * C].reshape(MAXV, MAXP, C)
    coors = coor[: MAXV * 3].reshape(MAXV, 3)
    return voxels, coors, npv[:MAXV], vnum[0]


# speedup vs baseline: 10.4427x; 10.4427x over previous
import dataclasses
import functools

import jax
import jax.numpy as jnp
from jax import lax
from jax.experimental import pallas as pl
from jax.experimental.pallas import tpu as pltpu
from jax.experimental.pallas import tpu_sc as plsc

GX, GY = 432, 496
NCELL = GX * GY
MAXV, MAXP, C = 20000, 32, 4
N = 200000
NT = 16
CPT = NCELL // NT
NCV = CPT // 16
CAP = 11776

VOX_WORDS = 2560512
VOXDUMPW = 2560480
COOR_WORDS = 60160
NPV_WORDS = 20096
CN_ROWS = 20688

VS0, VS1, VS2 = 0.16, 0.16, 4.0
PR0, PR1, PR2 = 0.0, -39.68, -3.0
CNTMASK = 0x3FFFF

_MESH = plsc.VectorSubcoreMesh(core_axis_name="c", subcore_axis_name="s",
                               num_cores=1)
_CP = pltpu.CompilerParams()
if "needs_layout_passes" in pltpu.CompilerParams.__dataclass_fields__:
    _CP = dataclasses.replace(_CP, needs_layout_passes=False)

NFULL, TAILP = 97, 1344


def _floor_div(q):
    ti = q.astype(jnp.int32)
    return ti - (ti.astype(jnp.float32) > q).astype(jnp.int32)


@functools.partial(
    pl.kernel,
    out_type=[
        jax.ShapeDtypeStruct((VOX_WORDS,), jnp.float32),
        jax.ShapeDtypeStruct((COOR_WORDS,), jnp.int32),
        jax.ShapeDtypeStruct((NPV_WORDS,), jnp.int32),
        jax.ShapeDtypeStruct((16,), jnp.int32),
    ],
    mesh=_MESH,
    compiler_params=_CP,
    scratch_types=[
        pltpu.HBM((CN_ROWS * 8,), jnp.int32),
        pltpu.HBM((256,), jnp.int32),
        pltpu.VMEM((8192,), jnp.float32),
        pltpu.VMEM((CPT,), jnp.int32),
        pltpu.VMEM((CAP + 16,), jnp.int32),
        pltpu.VMEM((CAP + 16,), jnp.float32),
        pltpu.VMEM((CAP + 16,), jnp.float32),
        pltpu.VMEM((CAP + 16,), jnp.float32),
        pltpu.VMEM((CAP + 16,), jnp.float32),
        pltpu.VMEM((32768,), jnp.float32),
        pltpu.VMEM((2176,), jnp.int32),
        pltpu.VMEM((2048,), jnp.int32),
        pltpu.VMEM((8192,), jnp.int32),
        pltpu.VMEM((2048,), jnp.float32),
        pltpu.VMEM((16,), jnp.int32),
        pltpu.VMEM((16,), jnp.int32),
        pltpu.VMEM((16,), jnp.float32),
        pltpu.SemaphoreType.DMA,
    ],
)
def _vox_kernel(pts_hbm, vox_out, coor_out, npv_out, vnum_out,
                cn_hbm, occ_hbm,
                pbuf, hcell, ks_s, ks_x, ks_y, ks_z, ks_w,
                wstage, cnst, ostage, rbuf, zf, b16, ovi, ovv, sem):
    t = lax.axis_index("s")
    lane = lax.iota(jnp.int32, 16)
    ones = jnp.ones((16,), jnp.int32)
    zeros16 = jnp.zeros((16,), jnp.int32)
    zf16 = jnp.zeros((16,), jnp.float32)
    lo = t * CPT

    def lin_of(j):
        idx = (j * 16 + lane) * 4
        x = plsc.load_gather(pbuf, [idx])
        y = plsc.load_gather(pbuf, [idx + 1])
        z = plsc.load_gather(pbuf, [idx + 2])
        cx = _floor_div((x - PR0) / jnp.float32(VS0))
        cy = _floor_div((y - PR1) / jnp.float32(VS1))
        cz = _floor_div((z - PR2) / jnp.float32(VS2))
        valid = ((cx >= 0) & (cx < GX) & (cy >= 0) & (cy < GY) & (cz == 0))
        return jnp.where(valid, cy * GX + cx, NCELL), x, y, z, idx

    def hz(i, _):
        hcell[pl.ds(i * 16, 16)] = zeros16
        return 0

    lax.fori_loop(0, NCV, hz, 0)

    def p1_chunk(nvec):
        def body(j, _):
            lin, _x, _y, _z, _i = lin_of(j)
            inr = (lin >= lo) & (lin < lo + CPT)
            cell = jnp.where(inr, lin - lo, 0)
            plsc.addupdate_scatter(hcell, [cell], ones, mask=inr)
            return 0

        lax.fori_loop(0, nvec, body, 0)

    def p1(c0, _):
        pltpu.sync_copy(pts_hbm.at[pl.ds(c0 * 8192, 8192)], pbuf)
        p1_chunk(128)
        return 0

    lax.fori_loop(0, NFULL, p1, 0)
    pltpu.sync_copy(pts_hbm.at[pl.ds(NFULL * 8192, TAILP * 4)],
                    pbuf.at[pl.ds(0, TAILP * 4)])
    p1_chunk(TAILP // 16)

    def p2(i, carry):
        h = hcell[pl.ds(i * 16, 16)]
        occ = (h > 0).astype(jnp.int32)
        excl = plsc.cumsum(occ) - occ + carry
        hcell[pl.ds(i * 16, 16)] = excl << 18
        return carry + jnp.sum(occ)

    occ_t = lax.fori_loop(0, NCV, p2, jnp.int32(0))
    b16[...] = jnp.full((16,), occ_t, jnp.int32)
    pltpu.sync_copy(b16, occ_hbm.at[pl.ds(pl.multiple_of(t * 16, 8), 16)])

    def zb(i, _):
        zf[pl.ds(i * 16, 16)] = zf16
        return 0

    lax.fori_loop(0, 128, zb, 0)
    vz = pl.multiple_of(t * 160032, 8)
    for k in range(78):
        pltpu.sync_copy(zf, vox_out.at[pl.ds(vz + k * 2048, 2048)])
    pltpu.sync_copy(zf.at[pl.ds(0, 288)],
                    vox_out.at[pl.ds(vz + 78 * 2048, 288)])

    plsc.subcore_barrier()
    pltpu.sync_copy(occ_hbm, cnst.at[pl.ds(0, 256)])
    occv = plsc.load_gather(cnst, [lane * 16])
    rb = jnp.sum(jnp.where(lane < t, occv, 0))
    total_occ = jnp.sum(occv)
    vn = jnp.minimum(total_occ, MAXV)
    nout = jnp.clip(jnp.minimum(occ_t, MAXV - rb), 0, MAXV)

    @pl.when(t == 0)
    def _():
        b16[...] = jnp.full((16,), vn, jnp.int32)
        pltpu.sync_copy(b16, vnum_out)

    def emit_chunk(nvec, kc0, append):
        def body(j, kc):
            lin, x, y, z, idx = lin_of(j)
            w = plsc.load_gather(pbuf, [idx + 3])
            inr = (lin >= lo) & (lin < lo + CPT)
            cell = jnp.where(inr, lin - lo, 0)
            h = plsc.load_gather(hcell, [cell], mask=inr)
            prior, _u = plsc.scan_count(cell, mask=inr)
            pos = (h & CNTMASK) + prior - 1
            lr = lax.shift_right_logical(h, 18)
            plsc.addupdate_scatter(hcell, [cell], ones, mask=inr)
            keep = inr & (pos < MAXP) & (lr < nout)
            rel = lr * MAXP + pos
            return append(kc, keep, rel, x, y, z, w)

        return lax.fori_loop(0, nvec, body, kc0)

    def emit_pass(kc0, append):
        def pc(c0, kc):
            pltpu.sync_copy(pts_hbm.at[pl.ds(c0 * 8192, 8192)], pbuf)
            return emit_chunk(128, kc, append)

        kc = lax.fori_loop(0, NFULL, pc, kc0)
        pltpu.sync_copy(pts_hbm.at[pl.ds(NFULL * 8192, TAILP * 4)],
                        pbuf.at[pl.ds(0, TAILP * 4)])
        return emit_chunk(TAILP // 16, kc, append)

    def append_list(kc, keep, rel, x, y, z, w):
        pc2 = plsc.cumsum(keep.astype(jnp.int32))
        incap = keep & ((kc + pc2 - 1) < CAP)
        base = jnp.minimum(kc, CAP)
        plsc.store_compressed(ks_s.at[pl.ds(base, 16)], rel, mask=incap)
        plsc.store_compressed(ks_x.at[pl.ds(base, 16)], x, mask=incap)
        plsc.store_compressed(ks_y.at[pl.ds(base, 16)], y, mask=incap)
        plsc.store_compressed(ks_z.at[pl.ds(base, 16)], z, mask=incap)
        plsc.store_compressed(ks_w.at[pl.ds(base, 16)], w, mask=incap)
        return kc + jnp.sum(keep.astype(jnp.int32))

    kept = emit_pass(jnp.int32(0), append_list)

    kcl = jnp.minimum(kept, CAP)
    nwin = (nout + 255) // 256

    def p4(w, _):
        def wz(i, _):
            wstage[pl.ds(i * 16, 16)] = zf16
            return 0

        lax.fori_loop(0, 2048, wz, 0)

        def place(i, _):
            m = (i * 16 + lane) < kcl
            sl = ks_s[pl.ds(i * 16, 16)]
            relw = sl - w * 8192
            inw = m & (relw >= 0) & (relw < 8192)
            off = jnp.where(inw, relw * 4, 0)
            plsc.store_scatter(wstage, [off], ks_x[pl.ds(i * 16, 16)],
                               mask=inw)
            plsc.store_scatter(wstage, [off + 1], ks_y[pl.ds(i * 16, 16)],
                               mask=inw)
            plsc.store_scatter(wstage, [off + 2], ks_z[pl.ds(i * 16, 16)],
                               mask=inw)
            plsc.store_scatter(wstage, [off + 3], ks_w[pl.ds(i * 16, 16)],
                               mask=inw)
            return 0

        lax.fori_loop(0, (kcl + 15) // 16, place, 0)
        rows = jnp.minimum(nout - w * 256, 256)
        dst = pl.multiple_of((rb + w * 256) * 128, 8)

        @pl.when(rows == 256)
        def _():
            pltpu.sync_copy(wstage, vox_out.at[pl.ds(dst, 32768)])

        @pl.when(rows < 256)
        def _():
            def f16(q, _):
                pltpu.sync_copy(
                    wstage.at[pl.ds(pl.multiple_of(q * 2048, 8), 2048)],
                    vox_out.at[pl.ds(pl.multiple_of(dst + q * 2048, 8),
                                     2048)])
                return 0

            lax.fori_loop(0, rows // 16, f16, 0)
            r0 = rows // 16 * 16

            def f1(q, _):
                pltpu.sync_copy(
                    wstage.at[pl.ds(pl.multiple_of((r0 + q) * 128, 8), 128)],
                    vox_out.at[pl.ds(pl.multiple_of(dst + (r0 + q) * 128, 8),
                                     128)])
                return 0

            lax.fori_loop(0, rows - r0, f1, 0)

        return 0

    lax.fori_loop(0, nwin, p4, 0)

    def p5(i, cw):
        c0 = i * 16 + lane
        h = hcell[pl.ds(i * 16, 16)]
        cnt = h & CNTMASK
        lr = lax.shift_right_logical(h, 18)
        ok = (cnt > 0) & (lr < nout)
        g = lo + c0
        yv = g // GX
        xv = g - yv * GX
        off = jnp.where(ok, (lr - cw * 256) * 8, 2168)
        plsc.store_scatter(cnst, [off], zeros16, mask=ok)
        plsc.store_scatter(cnst, [off + 1], yv, mask=ok)
        plsc.store_scatter(cnst, [off + 2], xv, mask=ok)
        plsc.store_scatter(cnst, [off + 3], jnp.minimum(cnt, MAXP), mask=ok)
        hi = jnp.max(jnp.where(ok, lr, 0))
        crossed = hi >= (cw + 1) * 256

        @pl.when(crossed)
        def _():
            pltpu.sync_copy(
                cnst.at[pl.ds(0, 2048)],
                cn_hbm.at[pl.ds(pl.multiple_of((rb + cw * 256) * 8, 8),
                                2048)])
            for q in range(8):
                cnst[pl.ds(q * 16, 16)] = cnst[pl.ds(2048 + q * 16, 16)]

        return jnp.where(crossed, cw + 1, cw)

    cw = lax.fori_loop(0, NCV, p5, jnp.int32(0))
    rem = jnp.maximum(nout - cw * 256, 0)

    def fr16(q, _):
        pltpu.sync_copy(
            cnst.at[pl.ds(pl.multiple_of(q * 128, 8), 128)],
            cn_hbm.at[pl.ds(pl.multiple_of((rb + cw * 256 + q * 16) * 8, 8),
                            128)])
        return 0

    lax.fori_loop(0, rem // 16, fr16, 0)
    rr0 = rem // 16 * 16

    def fr1(q, _):
        pltpu.sync_copy(
            cnst.at[pl.ds(pl.multiple_of((rr0 + q) * 8, 8), 8)],
            cn_hbm.at[pl.ds(pl.multiple_of((rb + cw * 256 + rr0 + q) * 8, 8),
                            8)])
        return 0

    lax.fori_loop(0, rem - rr0, fr1, 0)

    @pl.when(kept > CAP)
    def _():
        def clr(i, _):
            h = hcell[pl.ds(i * 16, 16)]
            hcell[pl.ds(i * 16, 16)] = h & ~CNTMASK
            return 0

        lax.fori_loop(0, NCV, clr, 0)

        def append_ovf(kc, keep, rel, x, y, z, w):
            pc2 = plsc.cumsum(keep.astype(jnp.int32))
            ovf = keep & ((kc + pc2 - 1) >= CAP)

            @pl.when(jnp.sum(ovf.astype(jnp.int32)) > 0)
            def _():
                base = (rb * 128) + rel * 4
                for comp, val in ((0, x), (1, y), (2, z), (3, w)):
                    ovi[...] = jnp.where(ovf, base + comp, VOXDUMPW)
                    ovv[...] = val
                    pltpu.async_copy(ovv, vox_out.at[ovi], sem).wait()

            return kc + jnp.sum(keep.astype(jnp.int32))

        emit_pass(jnp.int32(0), append_ovf)

    plsc.subcore_barrier()

    for c in range(30):
        @pl.when(t == c % NT)
        def _(c=c):
            nw = 2048 if c < 29 else 608
            w0 = c * 2048
            row0 = w0 // 3
            pltpu.sync_copy(cn_hbm.at[pl.ds(row0 * 8, 5504)],
                            rbuf.at[pl.ds(0, 5504)])

            def rp(j, _):
                wd = w0 + j * 16 + lane
                r = wd // 3
                src = (r - row0) * 8 + (wd - r * 3)
                v = plsc.load_gather(rbuf, [src])
                ostage[pl.ds(j * 16, 16)] = jnp.where(r < vn, v, -1)
                return 0

            lax.fori_loop(0, nw // 16, rp, 0)
            pltpu.sync_copy(ostage.at[pl.ds(0, nw)],
                            coor_out.at[pl.ds(w0, nw)])

    for c in range(20):
        @pl.when(t == c % NT)
        def _(c=c):
            nw = 1024 if c < 19 else 544
            w0 = c * 1024
            pltpu.sync_copy(cn_hbm.at[pl.ds(w0 * 8, 8192)], rbuf)

            def rp(j, _):
                wd = w0 + j * 16 + lane
                src = (wd - w0) * 8 + 3
                v = plsc.load_gather(rbuf, [src])
                ostage[pl.ds(j * 16, 16)] = jnp.where(wd < vn, v, 0)
                return 0

            lax.fori_loop(0, nw // 16, rp, 0)
            pltpu.sync_copy(ostage.at[pl.ds(0, nw)],
                            npv_out.at[pl.ds(w0, nw)])


def kernel(points):
    pts_flat = points.reshape(-1)
    vox, coor, npv, vnum = _vox_kernel(pts_flat)
    voxels = vox[: MAXV * MAXP * C].reshape(MAXV, MAXP, C)
    coors = coor[: MAXV * 3].reshape(MAXV, 3)
    return voxels, coors, npv[:MAXV], vnum[0]


# speedup vs baseline: 11.7673x; 1.1268x over previous
import dataclasses
import functools

import jax
import jax.numpy as jnp
from jax import lax
from jax.experimental import pallas as pl
from jax.experimental.pallas import tpu as pltpu
from jax.experimental.pallas import tpu_sc as plsc

GX, GY = 432, 496
NCELL = GX * GY
MAXV, MAXP, C = 20000, 32, 4
N = 200000
NT = 16
CPT = NCELL // NT
NCV = CPT // 16
CAP = 11776

VOX_WORDS = 2560512
VOXDUMPW = 2560480
COOR_WORDS = 60160
NPV_WORDS = 20096
CN_ROWS = 20688

VS0, VS1, VS2 = 0.16, 0.16, 4.0
PR0, PR1, PR2 = 0.0, -39.68, -3.0
CNTMASK = 0x3FFFF

_MESH = plsc.VectorSubcoreMesh(core_axis_name="c", subcore_axis_name="s",
                               num_cores=1)
_CP = pltpu.CompilerParams()
if "needs_layout_passes" in pltpu.CompilerParams.__dataclass_fields__:
    _CP = dataclasses.replace(_CP, needs_layout_passes=False)

NFULL, TAILP = 97, 1344


def _floor_div(q):
    ti = q.astype(jnp.int32)
    return ti - (ti.astype(jnp.float32) > q).astype(jnp.int32)


@functools.partial(
    pl.kernel,
    out_type=[
        jax.ShapeDtypeStruct((VOX_WORDS,), jnp.float32),
        jax.ShapeDtypeStruct((COOR_WORDS,), jnp.int32),
        jax.ShapeDtypeStruct((NPV_WORDS,), jnp.int32),
        jax.ShapeDtypeStruct((16,), jnp.int32),
    ],
    mesh=_MESH,
    compiler_params=_CP,
    scratch_types=[
        pltpu.HBM((CN_ROWS * 8,), jnp.int32),
        pltpu.HBM((256,), jnp.int32),
        pltpu.HBM((200704,), jnp.int32),
        pltpu.VMEM((8192,), jnp.float32),
        pltpu.VMEM((CPT,), jnp.int32),
        pltpu.VMEM((CAP + 16,), jnp.int32),
        pltpu.VMEM((CAP + 16,), jnp.float32),
        pltpu.VMEM((CAP + 16,), jnp.float32),
        pltpu.VMEM((CAP + 16,), jnp.float32),
        pltpu.VMEM((CAP + 16,), jnp.float32),
        pltpu.VMEM((32768,), jnp.float32),
        pltpu.VMEM((2176,), jnp.int32),
        pltpu.VMEM((2048,), jnp.int32),
        pltpu.VMEM((8192,), jnp.int32),
        pltpu.VMEM((2048,), jnp.float32),
        pltpu.VMEM((16,), jnp.int32),
        pltpu.VMEM((16,), jnp.int32),
        pltpu.VMEM((16,), jnp.float32),
        pltpu.SemaphoreType.DMA,
    ],
)
def _vox_kernel(pts_hbm, vox_out, coor_out, npv_out, vnum_out,
                cn_hbm, occ_hbm, lin_hbm,
                pbuf, hcell, ks_s, ks_x, ks_y, ks_z, ks_w,
                wstage, cnst, ostage, rbuf, zf, b16, ovi, ovv, sem):
    t = lax.axis_index("s")
    lane = lax.iota(jnp.int32, 16)
    ones = jnp.ones((16,), jnp.int32)
    zeros16 = jnp.zeros((16,), jnp.int32)
    zf16 = jnp.zeros((16,), jnp.float32)
    lo = t * CPT

    def lin_of(j):
        idx = (j * 16 + lane) * 4
        x = plsc.load_gather(pbuf, [idx])
        y = plsc.load_gather(pbuf, [idx + 1])
        z = plsc.load_gather(pbuf, [idx + 2])
        cx = _floor_div((x - PR0) / jnp.float32(VS0))
        cy = _floor_div((y - PR1) / jnp.float32(VS1))
        cz = _floor_div((z - PR2) / jnp.float32(VS2))
        valid = ((cx >= 0) & (cx < GX) & (cy >= 0) & (cy < GY) & (cz == 0))
        return jnp.where(valid, cy * GX + cx, NCELL), x, y, z, idx

    def hz(i, _):
        hcell[pl.ds(i * 16, 16)] = zeros16
        return 0

    lax.fori_loop(0, NCV, hz, 0)

    p0s = pl.multiple_of(t * 12512, 8)

    def p0_chunk(c0, nvec):
        def body(j, _):
            lin, _x, _y, _z, _i = lin_of(j)
            ostage[pl.ds(j * 16, 16)] = lin
            return 0

        lax.fori_loop(0, nvec, body, 0)

    def p0(c0, _):
        pltpu.sync_copy(pts_hbm.at[pl.ds(pl.multiple_of((p0s + c0 * 2048) * 4,
                                                        8), 8192)], pbuf)
        p0_chunk(c0, 128)
        pltpu.sync_copy(ostage,
                        lin_hbm.at[pl.ds(pl.multiple_of(p0s + c0 * 2048, 8),
                                         2048)])
        return 0

    lax.fori_loop(0, 6, p0, 0)

    @pl.when(t < 15)
    def _():
        pltpu.sync_copy(pts_hbm.at[pl.ds(pl.multiple_of((p0s + 12288) * 4, 8),
                                         896)], pbuf.at[pl.ds(0, 896)])
        p0_chunk(6, 14)
        pltpu.sync_copy(ostage.at[pl.ds(0, 224)],
                        lin_hbm.at[pl.ds(pl.multiple_of(p0s + 12288, 8), 224)])

    @pl.when(t == 15)
    def _():
        pltpu.sync_copy(pts_hbm.at[pl.ds(pl.multiple_of((p0s + 12288) * 4, 8),
                                         128)], pbuf.at[pl.ds(0, 128)])
        p0_chunk(6, 2)
        pltpu.sync_copy(ostage.at[pl.ds(0, 32)],
                        lin_hbm.at[pl.ds(pl.multiple_of(p0s + 12288, 8), 32)])

    plsc.subcore_barrier()

    def p1_chunk(nvec):
        def body(j, _):
            lin = rbuf[pl.ds(j * 16, 16)]
            inr = (lin >= lo) & (lin < lo + CPT)
            cell = jnp.where(inr, lin - lo, 0)
            plsc.addupdate_scatter(hcell, [cell], ones, mask=inr)
            return 0

        lax.fori_loop(0, nvec, body, 0)

    def p1(c0, _):
        pltpu.sync_copy(lin_hbm.at[pl.ds(c0 * 8192, 8192)], rbuf)
        p1_chunk(512)
        return 0

    lax.fori_loop(0, 24, p1, 0)
    pltpu.sync_copy(lin_hbm.at[pl.ds(24 * 8192, 3392)],
                    rbuf.at[pl.ds(0, 3392)])
    p1_chunk(212)

    def p2(i, carry):
        h = hcell[pl.ds(i * 16, 16)]
        occ = (h > 0).astype(jnp.int32)
        excl = plsc.cumsum(occ) - occ + carry
        hcell[pl.ds(i * 16, 16)] = excl << 18
        return carry + jnp.sum(occ)

    occ_t = lax.fori_loop(0, NCV, p2, jnp.int32(0))
    b16[...] = jnp.full((16,), occ_t, jnp.int32)
    pltpu.sync_copy(b16, occ_hbm.at[pl.ds(pl.multiple_of(t * 16, 8), 16)])

    def zb(i, _):
        zf[pl.ds(i * 16, 16)] = zf16
        return 0

    lax.fori_loop(0, 128, zb, 0)
    vz = pl.multiple_of(t * 160032, 8)
    for k in range(78):
        pltpu.sync_copy(zf, vox_out.at[pl.ds(vz + k * 2048, 2048)])
    pltpu.sync_copy(zf.at[pl.ds(0, 288)],
                    vox_out.at[pl.ds(vz + 78 * 2048, 288)])

    plsc.subcore_barrier()
    pltpu.sync_copy(occ_hbm, cnst.at[pl.ds(0, 256)])
    occv = plsc.load_gather(cnst, [lane * 16])
    rb = jnp.sum(jnp.where(lane < t, occv, 0))
    total_occ = jnp.sum(occv)
    vn = jnp.minimum(total_occ, MAXV)
    nout = jnp.clip(jnp.minimum(occ_t, MAXV - rb), 0, MAXV)

    @pl.when(t == 0)
    def _():
        b16[...] = jnp.full((16,), vn, jnp.int32)
        pltpu.sync_copy(b16, vnum_out)

    def emit_chunk(nvec, kc0, append):
        def body(j, kc):
            idx = (j * 16 + lane) * 4
            lin = rbuf[pl.ds(j * 16, 16)]
            x = plsc.load_gather(pbuf, [idx])
            y = plsc.load_gather(pbuf, [idx + 1])
            z = plsc.load_gather(pbuf, [idx + 2])
            w = plsc.load_gather(pbuf, [idx + 3])
            inr = (lin >= lo) & (lin < lo + CPT)
            cell = jnp.where(inr, lin - lo, 0)
            h = plsc.load_gather(hcell, [cell], mask=inr)
            prior, _u = plsc.scan_count(cell, mask=inr)
            pos = (h & CNTMASK) + prior - 1
            lr = lax.shift_right_logical(h, 18)
            plsc.addupdate_scatter(hcell, [cell], ones, mask=inr)
            keep = inr & (pos < MAXP) & (lr < nout)
            rel = lr * MAXP + pos
            return append(kc, keep, rel, x, y, z, w)

        return lax.fori_loop(0, nvec, body, kc0)

    def emit_pass(kc0, append):
        def pc(c0, kc):
            pltpu.sync_copy(pts_hbm.at[pl.ds(c0 * 8192, 8192)], pbuf)
            pltpu.sync_copy(lin_hbm.at[pl.ds(c0 * 2048, 2048)],
                            rbuf.at[pl.ds(0, 2048)])
            return emit_chunk(128, kc, append)

        kc = lax.fori_loop(0, NFULL, pc, kc0)
        pltpu.sync_copy(pts_hbm.at[pl.ds(NFULL * 8192, TAILP * 4)],
                        pbuf.at[pl.ds(0, TAILP * 4)])
        pltpu.sync_copy(lin_hbm.at[pl.ds(NFULL * 2048, TAILP)],
                        rbuf.at[pl.ds(0, TAILP)])
        return emit_chunk(TAILP // 16, kc, append)

    def append_list(kc, keep, rel, x, y, z, w):
        pc2 = plsc.cumsum(keep.astype(jnp.int32))
        incap = keep & ((kc + pc2 - 1) < CAP)
        base = jnp.minimum(kc, CAP)
        plsc.store_compressed(ks_s.at[pl.ds(base, 16)], rel, mask=incap)
        plsc.store_compressed(ks_x.at[pl.ds(base, 16)], x, mask=incap)
        plsc.store_compressed(ks_y.at[pl.ds(base, 16)], y, mask=incap)
        plsc.store_compressed(ks_z.at[pl.ds(base, 16)], z, mask=incap)
        plsc.store_compressed(ks_w.at[pl.ds(base, 16)], w, mask=incap)
        return kc + jnp.sum(keep.astype(jnp.int32))

    kept = emit_pass(jnp.int32(0), append_list)

    kcl = jnp.minimum(kept, CAP)
    nwin = (nout + 255) // 256

    def p4(w, _):
        def wz(i, _):
            wstage[pl.ds(i * 16, 16)] = zf16
            return 0

        lax.fori_loop(0, 2048, wz, 0)

        def place(i, _):
            m = (i * 16 + lane) < kcl
            sl = ks_s[pl.ds(i * 16, 16)]
            relw = sl - w * 8192
            inw = m & (relw >= 0) & (relw < 8192)
            off = jnp.where(inw, relw * 4, 0)
            plsc.store_scatter(wstage, [off], ks_x[pl.ds(i * 16, 16)],
                               mask=inw)
            plsc.store_scatter(wstage, [off + 1], ks_y[pl.ds(i * 16, 16)],
                               mask=inw)
            plsc.store_scatter(wstage, [off + 2], ks_z[pl.ds(i * 16, 16)],
                               mask=inw)
            plsc.store_scatter(wstage, [off + 3], ks_w[pl.ds(i * 16, 16)],
                               mask=inw)
            return 0

        lax.fori_loop(0, (kcl + 15) // 16, place, 0)
        rows = jnp.minimum(nout - w * 256, 256)
        dst = pl.multiple_of((rb + w * 256) * 128, 8)

        @pl.when(rows == 256)
        def _():
            pltpu.sync_copy(wstage, vox_out.at[pl.ds(dst, 32768)])

        @pl.when(rows < 256)
        def _():
            def f16(q, _):
                pltpu.sync_copy(
                    wstage.at[pl.ds(pl.multiple_of(q * 2048, 8), 2048)],
                    vox_out.at[pl.ds(pl.multiple_of(dst + q * 2048, 8),
                                     2048)])
                return 0

            lax.fori_loop(0, rows // 16, f16, 0)
            r0 = rows // 16 * 16

            def f1(q, _):
                pltpu.sync_copy(
                    wstage.at[pl.ds(pl.multiple_of((r0 + q) * 128, 8), 128)],
                    vox_out.at[pl.ds(pl.multiple_of(dst + (r0 + q) * 128, 8),
                                     128)])
                return 0

            lax.fori_loop(0, rows - r0, f1, 0)

        return 0

    lax.fori_loop(0, nwin, p4, 0)

    def p5(i, cw):
        c0 = i * 16 + lane
        h = hcell[pl.ds(i * 16, 16)]
        cnt = h & CNTMASK
        lr = lax.shift_right_logical(h, 18)
        ok = (cnt > 0) & (lr < nout)
        g = lo + c0
        yv = g // GX
        xv = g - yv * GX
        off = jnp.where(ok, (lr - cw * 256) * 8, 2168)
        plsc.store_scatter(cnst, [off], zeros16, mask=ok)
        plsc.store_scatter(cnst, [off + 1], yv, mask=ok)
        plsc.store_scatter(cnst, [off + 2], xv, mask=ok)
        plsc.store_scatter(cnst, [off + 3], jnp.minimum(cnt, MAXP), mask=ok)
        hi = jnp.max(jnp.where(ok, lr, 0))
        crossed = hi >= (cw + 1) * 256

        @pl.when(crossed)
        def _():
            pltpu.sync_copy(
                cnst.at[pl.ds(0, 2048)],
                cn_hbm.at[pl.ds(pl.multiple_of((rb + cw * 256) * 8, 8),
                                2048)])
            for q in range(8):
                cnst[pl.ds(q * 16, 16)] = cnst[pl.ds(2048 + q * 16, 16)]

        return jnp.where(crossed, cw + 1, cw)

    cw = lax.fori_loop(0, NCV, p5, jnp.int32(0))
    rem = jnp.maximum(nout - cw * 256, 0)

    def fr16(q, _):
        pltpu.sync_copy(
            cnst.at[pl.ds(pl.multiple_of(q * 128, 8), 128)],
            cn_hbm.at[pl.ds(pl.multiple_of((rb + cw * 256 + q * 16) * 8, 8),
                            128)])
        return 0

    lax.fori_loop(0, rem // 16, fr16, 0)
    rr0 = rem // 16 * 16

    def fr1(q, _):
        pltpu.sync_copy(
            cnst.at[pl.ds(pl.multiple_of((rr0 + q) * 8, 8), 8)],
            cn_hbm.at[pl.ds(pl.multiple_of((rb + cw * 256 + rr0 + q) * 8, 8),
                            8)])
        return 0

    lax.fori_loop(0, rem - rr0, fr1, 0)

    @pl.when(kept > CAP)
    def _():
        def clr(i, _):
            h = hcell[pl.ds(i * 16, 16)]
            hcell[pl.ds(i * 16, 16)] = h & ~CNTMASK
            return 0

        lax.fori_loop(0, NCV, clr, 0)

        def append_ovf(kc, keep, rel, x, y, z, w):
            pc2 = plsc.cumsum(keep.astype(jnp.int32))
            ovf = keep & ((kc + pc2 - 1) >= CAP)

            @pl.when(jnp.sum(ovf.astype(jnp.int32)) > 0)
            def _():
                base = (rb * 128) + rel * 4
                for comp, val in ((0, x), (1, y), (2, z), (3, w)):
                    ovi[...] = jnp.where(ovf, base + comp, VOXDUMPW)
                    ovv[...] = val
                    pltpu.async_copy(ovv, vox_out.at[ovi], sem).wait()

            return kc + jnp.sum(keep.astype(jnp.int32))

        emit_pass(jnp.int32(0), append_ovf)

    plsc.subcore_barrier()

    for c in range(30):
        @pl.when(t == c % NT)
        def _(c=c):
            nw = 2048 if c < 29 else 608
            w0 = c * 2048
            row0 = w0 // 3
            pltpu.sync_copy(cn_hbm.at[pl.ds(row0 * 8, 5504)],
                            rbuf.at[pl.ds(0, 5504)])

            def rp(j, _):
                wd = w0 + j * 16 + lane
                r = wd // 3
                src = (r - row0) * 8 + (wd - r * 3)
                v = plsc.load_gather(rbuf, [src])
                ostage[pl.ds(j * 16, 16)] = jnp.where(r < vn, v, -1)
                return 0

            lax.fori_loop(0, nw // 16, rp, 0)
            pltpu.sync_copy(ostage.at[pl.ds(0, nw)],
                            coor_out.at[pl.ds(w0, nw)])

    for c in range(20):
        @pl.when(t == c % NT)
        def _(c=c):
            nw = 1024 if c < 19 else 544
            w0 = c * 1024
            pltpu.sync_copy(cn_hbm.at[pl.ds(w0 * 8, 8192)], rbuf)

            def rp(j, _):
                wd = w0 + j * 16 + lane
                src = (wd - w0) * 8 + 3
                v = plsc.load_gather(rbuf, [src])
                ostage[pl.ds(j * 16, 16)] = jnp.where(wd < vn, v, 0)
                return 0

            lax.fori_loop(0, nw // 16, rp, 0)
            pltpu.sync_copy(ostage.at[pl.ds(0, nw)],
                            npv_out.at[pl.ds(w0, nw)])


def kernel(points):
    pts_flat = points.reshape(-1)
    vox, coor, npv, vnum = _vox_kernel(pts_flat)
    voxels = vox[: MAXV * MAXP * C].reshape(MAXV, MAXP, C)
    coors = coor[: MAXV * 3].reshape(MAXV, 3)
    return voxels, coors, npv[:MAXV], vnum[0]


# speedup vs baseline: 13.1160x; 1.1146x over previous
import dataclasses
import functools

import jax
import jax.numpy as jnp
from jax import lax
from jax.experimental import pallas as pl
from jax.experimental.pallas import tpu as pltpu
from jax.experimental.pallas import tpu_sc as plsc

GX, GY = 432, 496
NCELL = GX * GY
MAXV, MAXP, C = 20000, 32, 4
N = 200000
NT = 16
CPT = NCELL // NT
NCV = CPT // 16
CAP = 8192

VOX_WORDS = 2560512
VOXDUMPW = 2560480
COOR_WORDS = 60160
NPV_WORDS = 20096
CN_ROWS = 20688

VS0, VS1, VS2 = 0.16, 0.16, 4.0
PR0, PR1, PR2 = 0.0, -39.68, -3.0
CNTMASK = 0x3FFFF

_MESH = plsc.VectorSubcoreMesh(core_axis_name="c", subcore_axis_name="s",
                               num_cores=1)
_CP = pltpu.CompilerParams()
if "needs_layout_passes" in pltpu.CompilerParams.__dataclass_fields__:
    _CP = dataclasses.replace(_CP, needs_layout_passes=False)

NFULL, TAILP = 97, 1344


def _floor_div(q):
    ti = q.astype(jnp.int32)
    return ti - (ti.astype(jnp.float32) > q).astype(jnp.int32)


@functools.partial(
    pl.kernel,
    out_type=[
        jax.ShapeDtypeStruct((VOX_WORDS,), jnp.float32),
        jax.ShapeDtypeStruct((COOR_WORDS,), jnp.int32),
        jax.ShapeDtypeStruct((NPV_WORDS,), jnp.int32),
        jax.ShapeDtypeStruct((16,), jnp.int32),
    ],
    mesh=_MESH,
    compiler_params=_CP,
    scratch_types=[
        pltpu.HBM((CN_ROWS * 8,), jnp.int32),
        pltpu.HBM((256,), jnp.int32),
        pltpu.HBM((200704,), jnp.int32),
        pltpu.VMEM((8192,), jnp.float32),
        pltpu.VMEM((8192,), jnp.float32),
        pltpu.VMEM((CPT,), jnp.int32),
        pltpu.VMEM((CAP + 16,), jnp.int32),
        pltpu.VMEM((CAP + 16,), jnp.float32),
        pltpu.VMEM((CAP + 16,), jnp.float32),
        pltpu.VMEM((CAP + 16,), jnp.float32),
        pltpu.VMEM((CAP + 16,), jnp.float32),
        pltpu.VMEM((32768,), jnp.float32),
        pltpu.VMEM((2176,), jnp.int32),
        pltpu.VMEM((2048,), jnp.int32),
        pltpu.VMEM((8192,), jnp.int32),
        pltpu.VMEM((2048,), jnp.float32),
        pltpu.VMEM((16,), jnp.int32),
        pltpu.VMEM((16,), jnp.int32),
        pltpu.VMEM((16,), jnp.float32),
        pltpu.SemaphoreType.DMA,
    ],
)
def _vox_kernel(pts_hbm, vox_out, coor_out, npv_out, vnum_out,
                cn_hbm, occ_hbm, lin_hbm,
                pbuf, pbuf2, hcell, ks_s, ks_x, ks_y, ks_z, ks_w,
                wstage, cnst, ostage, rbuf, zf, b16, ovi, ovv, sem):
    t = lax.axis_index("s")
    lane = lax.iota(jnp.int32, 16)
    ones = jnp.ones((16,), jnp.int32)
    zeros16 = jnp.zeros((16,), jnp.int32)
    zf16 = jnp.zeros((16,), jnp.float32)
    lo = t * CPT

    def lin_of(j):
        idx = (j * 16 + lane) * 4
        x = plsc.load_gather(pbuf, [idx])
        y = plsc.load_gather(pbuf, [idx + 1])
        z = plsc.load_gather(pbuf, [idx + 2])
        cx = _floor_div((x - PR0) / jnp.float32(VS0))
        cy = _floor_div((y - PR1) / jnp.float32(VS1))
        cz = _floor_div((z - PR2) / jnp.float32(VS2))
        valid = ((cx >= 0) & (cx < GX) & (cy >= 0) & (cy < GY) & (cz == 0))
        return jnp.where(valid, cy * GX + cx, NCELL), x, y, z, idx

    def hz(i, _):
        hcell[pl.ds(i * 16, 16)] = zeros16
        return 0

    lax.fori_loop(0, NCV, hz, 0)

    p0s = pl.multiple_of(t * 12512, 8)

    def p0_chunk(c0, nvec):
        def body(j, _):
            lin, _x, _y, _z, _i = lin_of(j)
            ostage[pl.ds(j * 16, 16)] = lin
            return 0

        lax.fori_loop(0, nvec, body, 0)

    def p0(c0, _):
        pltpu.sync_copy(pts_hbm.at[pl.ds(pl.multiple_of((p0s + c0 * 2048) * 4,
                                                        8), 8192)], pbuf)
        p0_chunk(c0, 128)
        pltpu.sync_copy(ostage,
                        lin_hbm.at[pl.ds(pl.multiple_of(p0s + c0 * 2048, 8),
                                         2048)])
        return 0

    lax.fori_loop(0, 6, p0, 0)

    @pl.when(t < 15)
    def _():
        pltpu.sync_copy(pts_hbm.at[pl.ds(pl.multiple_of((p0s + 12288) * 4, 8),
                                         896)], pbuf.at[pl.ds(0, 896)])
        p0_chunk(6, 14)
        pltpu.sync_copy(ostage.at[pl.ds(0, 224)],
                        lin_hbm.at[pl.ds(pl.multiple_of(p0s + 12288, 8), 224)])

    @pl.when(t == 15)
    def _():
        pltpu.sync_copy(pts_hbm.at[pl.ds(pl.multiple_of((p0s + 12288) * 4, 8),
                                         128)], pbuf.at[pl.ds(0, 128)])
        p0_chunk(6, 2)
        pltpu.sync_copy(ostage.at[pl.ds(0, 32)],
                        lin_hbm.at[pl.ds(pl.multiple_of(p0s + 12288, 8), 32)])

    plsc.subcore_barrier()

    def p1_chunk(nvec, boff):
        def body(j, _):
            lin = rbuf[pl.ds(boff + j * 16, 16)]
            inr = (lin >= lo) & (lin < lo + CPT)
            cell = jnp.where(inr, lin - lo, 0)
            plsc.addupdate_scatter(hcell, [cell], ones, mask=inr)
            return 0

        lax.fori_loop(0, nvec, body, 0)

    def lin_cp(c0, half):
        return pltpu.async_copy(
            lin_hbm.at[pl.ds(pl.multiple_of(c0 * 4096, 8), 4096)],
            rbuf.at[pl.ds(half * 4096, 4096)], sem)

    lin_cp(0, 0)
    lin_cp(1, 1)

    def p1(k, _):
        for h in range(2):
            c = 2 * k + h
            pltpu.make_async_copy(
                lin_hbm.at[pl.ds(pl.multiple_of(c * 4096, 8), 4096)],
                rbuf.at[pl.ds(h * 4096, 4096)], sem).wait()
            p1_chunk(256, h * 4096)

            @pl.when(c + 2 < 48)
            def _(c=c, h=h):
                lin_cp(c + 2, h)

        return 0

    lax.fori_loop(0, 24, p1, 0)
    pltpu.sync_copy(lin_hbm.at[pl.ds(48 * 4096, 3392)],
                    rbuf.at[pl.ds(0, 3392)])
    p1_chunk(212, 0)

    def p2(i, carry):
        h = hcell[pl.ds(i * 16, 16)]
        occ = (h > 0).astype(jnp.int32)
        excl = plsc.cumsum(occ) - occ + carry
        hcell[pl.ds(i * 16, 16)] = excl << 18
        return carry + jnp.sum(occ)

    occ_t = lax.fori_loop(0, NCV, p2, jnp.int32(0))
    b16[...] = jnp.full((16,), occ_t, jnp.int32)
    pltpu.sync_copy(b16, occ_hbm.at[pl.ds(pl.multiple_of(t * 16, 8), 16)])

    def zb(i, _):
        zf[pl.ds(i * 16, 16)] = zf16
        return 0

    lax.fori_loop(0, 128, zb, 0)
    vz = pl.multiple_of(t * 160032, 8)
    for k in range(78):
        pltpu.sync_copy(zf, vox_out.at[pl.ds(vz + k * 2048, 2048)])
    pltpu.sync_copy(zf.at[pl.ds(0, 288)],
                    vox_out.at[pl.ds(vz + 78 * 2048, 288)])

    plsc.subcore_barrier()
    pltpu.sync_copy(occ_hbm, cnst.at[pl.ds(0, 256)])
    occv = plsc.load_gather(cnst, [lane * 16])
    rb = jnp.sum(jnp.where(lane < t, occv, 0))
    total_occ = jnp.sum(occv)
    vn = jnp.minimum(total_occ, MAXV)
    nout = jnp.clip(jnp.minimum(occ_t, MAXV - rb), 0, MAXV)

    @pl.when(t == 0)
    def _():
        b16[...] = jnp.full((16,), vn, jnp.int32)
        pltpu.sync_copy(b16, vnum_out)

    def emit_chunk(nvec, kc0, append, pb, loff):
        def body(j, kc):
            idx = (j * 16 + lane) * 4
            lin = rbuf[pl.ds(loff + j * 16, 16)]
            x = plsc.load_gather(pb, [idx])
            y = plsc.load_gather(pb, [idx + 1])
            z = plsc.load_gather(pb, [idx + 2])
            w = plsc.load_gather(pb, [idx + 3])
            inr = (lin >= lo) & (lin < lo + CPT)
            cell = jnp.where(inr, lin - lo, 0)
            h = plsc.load_gather(hcell, [cell], mask=inr)
            prior, _u = plsc.scan_count(cell, mask=inr)
            pos = (h & CNTMASK) + prior - 1
            lr = lax.shift_right_logical(h, 18)
            plsc.addupdate_scatter(hcell, [cell], ones, mask=inr)
            keep = inr & (pos < MAXP) & (lr < nout)
            rel = lr * MAXP + pos
            return append(kc, keep, rel, x, y, z, w)

        return lax.fori_loop(0, nvec, body, kc0)

    def emit_pass(kc0, append):
        pbufs = (pbuf, pbuf2)

        def pt_cp(c0, h):
            pltpu.async_copy(
                pts_hbm.at[pl.ds(pl.multiple_of(c0 * 8192, 8), 8192)],
                pbufs[h], sem)
            pltpu.async_copy(
                lin_hbm.at[pl.ds(pl.multiple_of(c0 * 2048, 8), 2048)],
                rbuf.at[pl.ds(h * 2048, 2048)], sem)

        def pt_wait(c0, h):
            pltpu.make_async_copy(
                pts_hbm.at[pl.ds(pl.multiple_of(c0 * 8192, 8), 8192)],
                pbufs[h], sem).wait()
            pltpu.make_async_copy(
                lin_hbm.at[pl.ds(pl.multiple_of(c0 * 2048, 8), 2048)],
                rbuf.at[pl.ds(h * 2048, 2048)], sem).wait()

        pt_cp(0, 0)
        pt_cp(1, 1)

        def pc(k, kc):
            for h in range(2):
                c = 2 * k + h
                pt_wait(c, h)
                kc = emit_chunk(128, kc, append, pbufs[h], h * 2048)

                @pl.when(c + 2 < 96)
                def _(c=c, h=h):
                    pt_cp(c + 2, h)

            return kc

        kc = lax.fori_loop(0, 48, pc, kc0)
        pltpu.sync_copy(pts_hbm.at[pl.ds(96 * 8192, 8192)], pbuf)
        pltpu.sync_copy(lin_hbm.at[pl.ds(96 * 2048, 2048)],
                        rbuf.at[pl.ds(0, 2048)])
        kc = emit_chunk(128, kc, append, pbuf, 0)
        pltpu.sync_copy(pts_hbm.at[pl.ds(NFULL * 8192, TAILP * 4)],
                        pbuf.at[pl.ds(0, TAILP * 4)])
        pltpu.sync_copy(lin_hbm.at[pl.ds(NFULL * 2048, TAILP)],
                        rbuf.at[pl.ds(0, TAILP)])
        return emit_chunk(TAILP // 16, kc, append, pbuf, 0)

    def append_list(kc, keep, rel, x, y, z, w):
        pc2 = plsc.cumsum(keep.astype(jnp.int32))
        incap = keep & ((kc + pc2 - 1) < CAP)
        base = jnp.minimum(kc, CAP)
        plsc.store_compressed(ks_s.at[pl.ds(base, 16)], rel, mask=incap)
        plsc.store_compressed(ks_x.at[pl.ds(base, 16)], x, mask=incap)
        plsc.store_compressed(ks_y.at[pl.ds(base, 16)], y, mask=incap)
        plsc.store_compressed(ks_z.at[pl.ds(base, 16)], z, mask=incap)
        plsc.store_compressed(ks_w.at[pl.ds(base, 16)], w, mask=incap)
        return kc + jnp.sum(keep.astype(jnp.int32))

    kept = emit_pass(jnp.int32(0), append_list)

    kcl = jnp.minimum(kept, CAP)
    nwin = (nout + 255) // 256

    def p4(w, _):
        def wz(i, _):
            wstage[pl.ds(i * 16, 16)] = zf16
            return 0

        lax.fori_loop(0, 2048, wz, 0)

        def place(i, _):
            m = (i * 16 + lane) < kcl
            sl = ks_s[pl.ds(i * 16, 16)]
            relw = sl - w * 8192
            inw = m & (relw >= 0) & (relw < 8192)
            off = jnp.where(inw, relw * 4, 0)
            plsc.store_scatter(wstage, [off], ks_x[pl.ds(i * 16, 16)],
                               mask=inw)
            plsc.store_scatter(wstage, [off + 1], ks_y[pl.ds(i * 16, 16)],
                               mask=inw)
            plsc.store_scatter(wstage, [off + 2], ks_z[pl.ds(i * 16, 16)],
                               mask=inw)
            plsc.store_scatter(wstage, [off + 3], ks_w[pl.ds(i * 16, 16)],
                               mask=inw)
            return 0

        lax.fori_loop(0, (kcl + 15) // 16, place, 0)
        rows = jnp.minimum(nout - w * 256, 256)
        dst = pl.multiple_of((rb + w * 256) * 128, 8)

        @pl.when(rows == 256)
        def _():
            pltpu.sync_copy(wstage, vox_out.at[pl.ds(dst, 32768)])

        @pl.when(rows < 256)
        def _():
            def f16(q, _):
                pltpu.sync_copy(
                    wstage.at[pl.ds(pl.multiple_of(q * 2048, 8), 2048)],
                    vox_out.at[pl.ds(pl.multiple_of(dst + q * 2048, 8),
                                     2048)])
                return 0

            lax.fori_loop(0, rows // 16, f16, 0)
            r0 = rows // 16 * 16

            def f1(q, _):
                pltpu.sync_copy(
                    wstage.at[pl.ds(pl.multiple_of((r0 + q) * 128, 8), 128)],
                    vox_out.at[pl.ds(pl.multiple_of(dst + (r0 + q) * 128, 8),
                                     128)])
                return 0

            lax.fori_loop(0, rows - r0, f1, 0)

        return 0

    lax.fori_loop(0, nwin, p4, 0)

    def p5(i, cw):
        c0 = i * 16 + lane
        h = hcell[pl.ds(i * 16, 16)]
        cnt = h & CNTMASK
        lr = lax.shift_right_logical(h, 18)
        ok = (cnt > 0) & (lr < nout)
        g = lo + c0
        yv = g // GX
        xv = g - yv * GX
        off = jnp.where(ok, (lr - cw * 256) * 8, 2168)
        plsc.store_scatter(cnst, [off], zeros16, mask=ok)
        plsc.store_scatter(cnst, [off + 1], yv, mask=ok)
        plsc.store_scatter(cnst, [off + 2], xv, mask=ok)
        plsc.store_scatter(cnst, [off + 3], jnp.minimum(cnt, MAXP), mask=ok)
        hi = jnp.max(jnp.where(ok, lr, 0))
        crossed = hi >= (cw + 1) * 256

        @pl.when(crossed)
        def _():
            pltpu.sync_copy(
                cnst.at[pl.ds(0, 2048)],
                cn_hbm.at[pl.ds(pl.multiple_of((rb + cw * 256) * 8, 8),
                                2048)])
            for q in range(8):
                cnst[pl.ds(q * 16, 16)] = cnst[pl.ds(2048 + q * 16, 16)]

        return jnp.where(crossed, cw + 1, cw)

    cw = lax.fori_loop(0, NCV, p5, jnp.int32(0))
    rem = jnp.maximum(nout - cw * 256, 0)

    def fr16(q, _):
        pltpu.sync_copy(
            cnst.at[pl.ds(pl.multiple_of(q * 128, 8), 128)],
            cn_hbm.at[pl.ds(pl.multiple_of((rb + cw * 256 + q * 16) * 8, 8),
                            128)])
        return 0

    lax.fori_loop(0, rem // 16, fr16, 0)
    rr0 = rem // 16 * 16

    def fr1(q, _):
        pltpu.sync_copy(
            cnst.at[pl.ds(pl.multiple_of((rr0 + q) * 8, 8), 8)],
            cn_hbm.at[pl.ds(pl.multiple_of((rb + cw * 256 + rr0 + q) * 8, 8),
                            8)])
        return 0

    lax.fori_loop(0, rem - rr0, fr1, 0)

    @pl.when(kept > CAP)
    def _():
        def clr(i, _):
            h = hcell[pl.ds(i * 16, 16)]
            hcell[pl.ds(i * 16, 16)] = h & ~CNTMASK
            return 0

        lax.fori_loop(0, NCV, clr, 0)

        def append_ovf(kc, keep, rel, x, y, z, w):
            pc2 = plsc.cumsum(keep.astype(jnp.int32))
            ovf = keep & ((kc + pc2 - 1) >= CAP)

            @pl.when(jnp.sum(ovf.astype(jnp.int32)) > 0)
            def _():
                base = (rb * 128) + rel * 4
                for comp, val in ((0, x), (1, y), (2, z), (3, w)):
                    ovi[...] = jnp.where(ovf, base + comp, VOXDUMPW)
                    ovv[...] = val
                    pltpu.async_copy(ovv, vox_out.at[ovi], sem).wait()

            return kc + jnp.sum(keep.astype(jnp.int32))

        emit_pass(jnp.int32(0), append_ovf)

    plsc.subcore_barrier()

    for c in range(30):
        @pl.when(t == c % NT)
        def _(c=c):
            nw = 2048 if c < 29 else 608
            w0 = c * 2048
            row0 = w0 // 3
            pltpu.sync_copy(cn_hbm.at[pl.ds(row0 * 8, 5504)],
                            rbuf.at[pl.ds(0, 5504)])

            def rp(j, _):
                wd = w0 + j * 16 + lane
                r = wd // 3
                src = (r - row0) * 8 + (wd - r * 3)
                v = plsc.load_gather(rbuf, [src])
                ostage[pl.ds(j * 16, 16)] = jnp.where(r < vn, v, -1)
                return 0

            lax.fori_loop(0, nw // 16, rp, 0)
            pltpu.sync_copy(ostage.at[pl.ds(0, nw)],
                            coor_out.at[pl.ds(w0, nw)])

    for c in range(20):
        @pl.when(t == c % NT)
        def _(c=c):
            nw = 1024 if c < 19 else 544
            w0 = c * 1024
            pltpu.sync_copy(cn_hbm.at[pl.ds(w0 * 8, 8192)], rbuf)

            def rp(j, _):
                wd = w0 + j * 16 + lane
                src = (wd - w0) * 8 + 3
                v = plsc.load_gather(rbuf, [src])
                ostage[pl.ds(j * 16, 16)] = jnp.where(wd < vn, v, 0)
                return 0

            lax.fori_loop(0, nw // 16, rp, 0)
            pltpu.sync_copy(ostage.at[pl.ds(0, nw)],
                            npv_out.at[pl.ds(w0, nw)])


def kernel(points):
    pts_flat = points.reshape(-1)
    vox, coor, npv, vnum = _vox_kernel(pts_flat)
    voxels = vox[: MAXV * MAXP * C].reshape(MAXV, MAXP, C)
    coors = coor[: MAXV * 3].reshape(MAXV, 3)
    return voxels, coors, npv[:MAXV], vnum[0]


# speedup vs baseline: 14.2402x; 1.0857x over previous
import dataclasses
import functools

import jax
import jax.numpy as jnp
from jax import lax
from jax.experimental import pallas as pl
from jax.experimental.pallas import tpu as pltpu
from jax.experimental.pallas import tpu_sc as plsc

GX, GY = 432, 496
NCELL = GX * GY
MAXV, MAXP, C = 20000, 32, 4
N = 200000
NT = 16
CPT = NCELL // NT
NCV = CPT // 16
CAP = 8192

VOX_WORDS = 2560512
VOXDUMPW = 2560480
COOR_WORDS = 60160
NPV_WORDS = 20096
CN_ROWS = 20688

VS0, VS1, VS2 = 0.16, 0.16, 4.0
PR0, PR1, PR2 = 0.0, -39.68, -3.0
CNTMASK = 0x3FFFF

_MESH = plsc.VectorSubcoreMesh(core_axis_name="c", subcore_axis_name="s",
                               num_cores=1)
_CP = pltpu.CompilerParams()
if "needs_layout_passes" in pltpu.CompilerParams.__dataclass_fields__:
    _CP = dataclasses.replace(_CP, needs_layout_passes=False)

NFULL, TAILP = 97, 1344


def _floor_div(q):
    ti = q.astype(jnp.int32)
    return ti - (ti.astype(jnp.float32) > q).astype(jnp.int32)


@functools.partial(
    pl.kernel,
    out_type=[
        jax.ShapeDtypeStruct((VOX_WORDS,), jnp.float32),
        jax.ShapeDtypeStruct((COOR_WORDS,), jnp.int32),
        jax.ShapeDtypeStruct((NPV_WORDS,), jnp.int32),
        jax.ShapeDtypeStruct((16,), jnp.int32),
    ],
    mesh=_MESH,
    compiler_params=_CP,
    scratch_types=[
        pltpu.HBM((CN_ROWS * 8,), jnp.int32),
        pltpu.HBM((256,), jnp.int32),
        pltpu.HBM((200704,), jnp.int32),
        pltpu.VMEM((8192,), jnp.float32),
        pltpu.VMEM((8192,), jnp.float32),
        pltpu.VMEM((CPT,), jnp.int32),
        pltpu.VMEM((CAP + 16,), jnp.int32),
        pltpu.VMEM((CAP + 16,), jnp.float32),
        pltpu.VMEM((CAP + 16,), jnp.float32),
        pltpu.VMEM((CAP + 16,), jnp.float32),
        pltpu.VMEM((CAP + 16,), jnp.float32),
        pltpu.VMEM((CAP + 16,), jnp.int32),
        pltpu.VMEM((32768,), jnp.float32),
        pltpu.VMEM((2176,), jnp.int32),
        pltpu.VMEM((2048,), jnp.int32),
        pltpu.VMEM((8192,), jnp.int32),
        pltpu.VMEM((2048,), jnp.float32),
        pltpu.VMEM((16,), jnp.int32),
        pltpu.VMEM((16,), jnp.int32),
        pltpu.VMEM((16,), jnp.float32),
        pltpu.SemaphoreType.DMA,
    ],
)
def _vox_kernel(pts_hbm, vox_out, coor_out, npv_out, vnum_out,
                cn_hbm, occ_hbm, lin_hbm,
                pbuf, pbuf2, hcell, ks_s, ks_x, ks_y, ks_z, ks_w, ks_i,
                wstage, cnst, ostage, rbuf, zf, b16, ovi, ovv, sem):
    t = lax.axis_index("s")
    lane = lax.iota(jnp.int32, 16)
    ones = jnp.ones((16,), jnp.int32)
    zeros16 = jnp.zeros((16,), jnp.int32)
    zf16 = jnp.zeros((16,), jnp.float32)
    lo = t * CPT

    def lin_of(j):
        idx = (j * 16 + lane) * 4
        x = plsc.load_gather(pbuf, [idx])
        y = plsc.load_gather(pbuf, [idx + 1])
        z = plsc.load_gather(pbuf, [idx + 2])
        cx = _floor_div((x - PR0) / jnp.float32(VS0))
        cy = _floor_div((y - PR1) / jnp.float32(VS1))
        cz = _floor_div((z - PR2) / jnp.float32(VS2))
        valid = ((cx >= 0) & (cx < GX) & (cy >= 0) & (cy < GY) & (cz == 0))
        return jnp.where(valid, cy * GX + cx, NCELL), x, y, z, idx

    def hz(i, _):
        hcell[pl.ds(i * 16, 16)] = zeros16
        return 0

    lax.fori_loop(0, NCV, hz, 0)

    p0s = pl.multiple_of(t * 12512, 8)

    def p0_chunk(c0, nvec):
        def body(j, _):
            lin, _x, _y, _z, _i = lin_of(j)
            ostage[pl.ds(j * 16, 16)] = lin
            return 0

        lax.fori_loop(0, nvec, body, 0)

    def p0(c0, _):
        pltpu.sync_copy(pts_hbm.at[pl.ds(pl.multiple_of((p0s + c0 * 2048) * 4,
                                                        8), 8192)], pbuf)
        p0_chunk(c0, 128)
        pltpu.sync_copy(ostage,
                        lin_hbm.at[pl.ds(pl.multiple_of(p0s + c0 * 2048, 8),
                                         2048)])
        return 0

    lax.fori_loop(0, 6, p0, 0)

    @pl.when(t < 15)
    def _():
        pltpu.sync_copy(pts_hbm.at[pl.ds(pl.multiple_of((p0s + 12288) * 4, 8),
                                         896)], pbuf.at[pl.ds(0, 896)])
        p0_chunk(6, 14)
        pltpu.sync_copy(ostage.at[pl.ds(0, 224)],
                        lin_hbm.at[pl.ds(pl.multiple_of(p0s + 12288, 8), 224)])

    @pl.when(t == 15)
    def _():
        pltpu.sync_copy(pts_hbm.at[pl.ds(pl.multiple_of((p0s + 12288) * 4, 8),
                                         128)], pbuf.at[pl.ds(0, 128)])
        p0_chunk(6, 2)
        pltpu.sync_copy(ostage.at[pl.ds(0, 32)],
                        lin_hbm.at[pl.ds(pl.multiple_of(p0s + 12288, 8), 32)])

    plsc.subcore_barrier()

    def p1_chunk(nvec, boff):
        def body(j, _):
            lin = rbuf[pl.ds(boff + j * 16, 16)]
            inr = (lin >= lo) & (lin < lo + CPT)
            cell = jnp.where(inr, lin - lo, 0)
            plsc.addupdate_scatter(hcell, [cell], ones, mask=inr)
            return 0

        lax.fori_loop(0, nvec, body, 0)

    def lin_cp(c0, half):
        return pltpu.async_copy(
            lin_hbm.at[pl.ds(pl.multiple_of(c0 * 4096, 8), 4096)],
            rbuf.at[pl.ds(half * 4096, 4096)], sem)

    lin_cp(0, 0)
    lin_cp(1, 1)

    def p1(k, _):
        for h in range(2):
            c = 2 * k + h
            pltpu.make_async_copy(
                lin_hbm.at[pl.ds(pl.multiple_of(c * 4096, 8), 4096)],
                rbuf.at[pl.ds(h * 4096, 4096)], sem).wait()
            p1_chunk(256, h * 4096)

            @pl.when(c + 2 < 48)
            def _(c=c, h=h):
                lin_cp(c + 2, h)

        return 0

    lax.fori_loop(0, 24, p1, 0)
    pltpu.sync_copy(lin_hbm.at[pl.ds(48 * 4096, 3392)],
                    rbuf.at[pl.ds(0, 3392)])
    p1_chunk(212, 0)

    def p2(i, carry):
        h = hcell[pl.ds(i * 16, 16)]
        occ = (h > 0).astype(jnp.int32)
        excl = plsc.cumsum(occ) - occ + carry
        hcell[pl.ds(i * 16, 16)] = excl << 18
        return carry + jnp.sum(occ)

    occ_t = lax.fori_loop(0, NCV, p2, jnp.int32(0))
    b16[...] = jnp.full((16,), occ_t, jnp.int32)
    pltpu.sync_copy(b16, occ_hbm.at[pl.ds(pl.multiple_of(t * 16, 8), 16)])

    def zb(i, _):
        zf[pl.ds(i * 16, 16)] = zf16
        return 0

    lax.fori_loop(0, 128, zb, 0)
    vz = pl.multiple_of(t * 160032, 8)
    for k in range(78):
        pltpu.sync_copy(zf, vox_out.at[pl.ds(vz + k * 2048, 2048)])
    pltpu.sync_copy(zf.at[pl.ds(0, 288)],
                    vox_out.at[pl.ds(vz + 78 * 2048, 288)])

    plsc.subcore_barrier()
    pltpu.sync_copy(occ_hbm, cnst.at[pl.ds(0, 256)])
    occv = plsc.load_gather(cnst, [lane * 16])
    rb = jnp.sum(jnp.where(lane < t, occv, 0))
    total_occ = jnp.sum(occv)
    vn = jnp.minimum(total_occ, MAXV)
    nout = jnp.clip(jnp.minimum(occ_t, MAXV - rb), 0, MAXV)

    @pl.when(t == 0)
    def _():
        b16[...] = jnp.full((16,), vn, jnp.int32)
        pltpu.sync_copy(b16, vnum_out)

    def emit_chunk(nvec, kc0, append, pb, loff):
        def body(j, kc):
            idx = (j * 16 + lane) * 4
            lin = rbuf[pl.ds(loff + j * 16, 16)]
            x = plsc.load_gather(pb, [idx])
            y = plsc.load_gather(pb, [idx + 1])
            z = plsc.load_gather(pb, [idx + 2])
            w = plsc.load_gather(pb, [idx + 3])
            inr = (lin >= lo) & (lin < lo + CPT)
            cell = jnp.where(inr, lin - lo, 0)
            h = plsc.load_gather(hcell, [cell], mask=inr)
            prior, _u = plsc.scan_count(cell, mask=inr)
            pos = (h & CNTMASK) + prior - 1
            lr = lax.shift_right_logical(h, 18)
            plsc.addupdate_scatter(hcell, [cell], ones, mask=inr)
            keep = inr & (pos < MAXP) & (lr < nout)
            rel = lr * MAXP + pos
            return append(kc, keep, rel, x, y, z, w)

        return lax.fori_loop(0, nvec, body, kc0)

    def emit_pass(kc0, append):
        pbufs = (pbuf, pbuf2)

        def pt_cp(c0, h):
            pltpu.async_copy(
                pts_hbm.at[pl.ds(pl.multiple_of(c0 * 8192, 8), 8192)],
                pbufs[h], sem)
            pltpu.async_copy(
                lin_hbm.at[pl.ds(pl.multiple_of(c0 * 2048, 8), 2048)],
                rbuf.at[pl.ds(h * 2048, 2048)], sem)

        def pt_wait(c0, h):
            pltpu.make_async_copy(
                pts_hbm.at[pl.ds(pl.multiple_of(c0 * 8192, 8), 8192)],
                pbufs[h], sem).wait()
            pltpu.make_async_copy(
                lin_hbm.at[pl.ds(pl.multiple_of(c0 * 2048, 8), 2048)],
                rbuf.at[pl.ds(h * 2048, 2048)], sem).wait()

        pt_cp(0, 0)
        pt_cp(1, 1)

        def pc(k, kc):
            for h in range(2):
                c = 2 * k + h
                pt_wait(c, h)
                kc = emit_chunk(128, kc, append, pbufs[h], h * 2048)

                @pl.when(c + 2 < 96)
                def _(c=c, h=h):
                    pt_cp(c + 2, h)

            return kc

        kc = lax.fori_loop(0, 48, pc, kc0)
        pltpu.sync_copy(pts_hbm.at[pl.ds(96 * 8192, 8192)], pbuf)
        pltpu.sync_copy(lin_hbm.at[pl.ds(96 * 2048, 2048)],
                        rbuf.at[pl.ds(0, 2048)])
        kc = emit_chunk(128, kc, append, pbuf, 0)
        pltpu.sync_copy(pts_hbm.at[pl.ds(NFULL * 8192, TAILP * 4)],
                        pbuf.at[pl.ds(0, TAILP * 4)])
        pltpu.sync_copy(lin_hbm.at[pl.ds(NFULL * 2048, TAILP)],
                        rbuf.at[pl.ds(0, TAILP)])
        return emit_chunk(TAILP // 16, kc, append, pbuf, 0)

    def append_list(kc, keep, rel, x, y, z, w):
        pc2 = plsc.cumsum(keep.astype(jnp.int32))
        incap = keep & ((kc + pc2 - 1) < CAP)
        base = jnp.minimum(kc, CAP)
        plsc.store_compressed(ks_s.at[pl.ds(base, 16)], rel, mask=incap)
        plsc.store_compressed(ks_x.at[pl.ds(base, 16)], x, mask=incap)
        plsc.store_compressed(ks_y.at[pl.ds(base, 16)], y, mask=incap)
        plsc.store_compressed(ks_z.at[pl.ds(base, 16)], z, mask=incap)
        plsc.store_compressed(ks_w.at[pl.ds(base, 16)], w, mask=incap)
        return kc + jnp.sum(keep.astype(jnp.int32))

    kept = emit_pass(jnp.int32(0), append_list)

    kcl = jnp.minimum(kept, CAP)
    nwin = (nout + 255) // 256
    for q in range(4):
        cnst[pl.ds(q * 16, 16)] = zeros16

    def wh(i, _):
        m = (i * 16 + lane) < kcl
        w = lax.shift_right_logical(ks_s[pl.ds(i * 16, 16)], 13)
        plsc.addupdate_scatter(cnst, [jnp.where(m, w, 0)], ones, mask=m)
        return 0

    nkv = (kcl + 15) // 16
    lax.fori_loop(0, nkv, wh, 0)
    carry = jnp.int32(0)
    for q in range(4):
        cv = cnst[pl.ds(q * 16, 16)]
        excl = plsc.cumsum(cv) - cv + carry
        cnst[pl.ds(64 + q * 16, 16)] = excl
        cnst[pl.ds(128 + q * 16, 16)] = excl
        carry = carry + jnp.sum(cv)

    def wscat(i, _):
        iv = i * 16 + lane
        m = iv < kcl
        w = jnp.where(m, lax.shift_right_logical(ks_s[pl.ds(i * 16, 16)], 13),
                      0)
        prior, _u = plsc.scan_count(w, mask=m)
        base = plsc.load_gather(cnst, [64 + w], mask=m)
        plsc.addupdate_scatter(cnst, [64 + w], ones, mask=m)
        dst = jnp.minimum(base + prior - 1, CAP)
        plsc.store_scatter(ks_i, [jnp.where(m, dst, CAP)], iv, mask=m)
        return 0

    lax.fori_loop(0, nkv, wscat, 0)

    def p4(w, _):
        def wz(i, _):
            wstage[pl.ds(i * 16, 16)] = zf16
            return 0

        lax.fori_loop(0, 2048, wz, 0)
        sv = plsc.load_gather(cnst, [jnp.full((16,), 128, jnp.int32) + w])
        ev = plsc.load_gather(cnst, [jnp.full((16,), 64, jnp.int32) + w])
        s0 = jnp.max(sv)
        e0 = jnp.max(ev)

        def place(i, _):
            p = s0 + i * 16 + lane
            m = p < e0
            ki = plsc.load_gather(ks_i, [jnp.minimum(p, CAP)], mask=m)
            sl = plsc.load_gather(ks_s, [ki], mask=m)
            off = (sl - w * 8192) * 4
            off = jnp.where(m, off, 0)
            plsc.store_scatter(wstage, [off],
                               plsc.load_gather(ks_x, [ki], mask=m), mask=m)
            plsc.store_scatter(wstage, [off + 1],
                               plsc.load_gather(ks_y, [ki], mask=m), mask=m)
            plsc.store_scatter(wstage, [off + 2],
                               plsc.load_gather(ks_z, [ki], mask=m), mask=m)
            plsc.store_scatter(wstage, [off + 3],
                               plsc.load_gather(ks_w, [ki], mask=m), mask=m)
            return 0

        lax.fori_loop(0, (e0 - s0 + 15) // 16, place, 0)
        rows = jnp.minimum(nout - w * 256, 256)
        dst = pl.multiple_of((rb + w * 256) * 128, 8)

        @pl.when(rows == 256)
        def _():
            pltpu.sync_copy(wstage, vox_out.at[pl.ds(dst, 32768)])

        @pl.when(rows < 256)
        def _():
            def f16(q, _):
                pltpu.sync_copy(
                    wstage.at[pl.ds(pl.multiple_of(q * 2048, 8), 2048)],
                    vox_out.at[pl.ds(pl.multiple_of(dst + q * 2048, 8),
                                     2048)])
                return 0

            lax.fori_loop(0, rows // 16, f16, 0)
            r0 = rows // 16 * 16

            def f1(q, _):
                pltpu.sync_copy(
                    wstage.at[pl.ds(pl.multiple_of((r0 + q) * 128, 8), 128)],
                    vox_out.at[pl.ds(pl.multiple_of(dst + (r0 + q) * 128, 8),
                                     128)])
                return 0

            lax.fori_loop(0, rows - r0, f1, 0)

        return 0

    lax.fori_loop(0, nwin, p4, 0)

    def p5(i, cw):
        c0 = i * 16 + lane
        h = hcell[pl.ds(i * 16, 16)]
        cnt = h & CNTMASK
        lr = lax.shift_right_logical(h, 18)
        ok = (cnt > 0) & (lr < nout)
        g = lo + c0
        yv = g // GX
        xv = g - yv * GX
        off = jnp.where(ok, (lr - cw * 256) * 8, 2168)
        plsc.store_scatter(cnst, [off], zeros16, mask=ok)
        plsc.store_scatter(cnst, [off + 1], yv, mask=ok)
        plsc.store_scatter(cnst, [off + 2], xv, mask=ok)
        plsc.store_scatter(cnst, [off + 3], jnp.minimum(cnt, MAXP), mask=ok)
        hi = jnp.max(jnp.where(ok, lr, 0))
        crossed = hi >= (cw + 1) * 256

        @pl.when(crossed)
        def _():
            pltpu.sync_copy(
                cnst.at[pl.ds(0, 2048)],
                cn_hbm.at[pl.ds(pl.multiple_of((rb + cw * 256) * 8, 8),
                                2048)])
            for q in range(8):
                cnst[pl.ds(q * 16, 16)] = cnst[pl.ds(2048 + q * 16, 16)]

        return jnp.where(crossed, cw + 1, cw)

    cw = lax.fori_loop(0, NCV, p5, jnp.int32(0))
    rem = jnp.maximum(nout - cw * 256, 0)

    def fr16(q, _):
        pltpu.sync_copy(
            cnst.at[pl.ds(pl.multiple_of(q * 128, 8), 128)],
            cn_hbm.at[pl.ds(pl.multiple_of((rb + cw * 256 + q * 16) * 8, 8),
                            128)])
        return 0

    lax.fori_loop(0, rem // 16, fr16, 0)
    rr0 = rem // 16 * 16

    def fr1(q, _):
        pltpu.sync_copy(
            cnst.at[pl.ds(pl.multiple_of((rr0 + q) * 8, 8), 8)],
            cn_hbm.at[pl.ds(pl.multiple_of((rb + cw * 256 + rr0 + q) * 8, 8),
                            8)])
        return 0

    lax.fori_loop(0, rem - rr0, fr1, 0)

    @pl.when(kept > CAP)
    def _():
        def clr(i, _):
            h = hcell[pl.ds(i * 16, 16)]
            hcell[pl.ds(i * 16, 16)] = h & ~CNTMASK
            return 0

        lax.fori_loop(0, NCV, clr, 0)

        def append_ovf(kc, keep, rel, x, y, z, w):
            pc2 = plsc.cumsum(keep.astype(jnp.int32))
            ovf = keep & ((kc + pc2 - 1) >= CAP)

            @pl.when(jnp.sum(ovf.astype(jnp.int32)) > 0)
            def _():
                base = (rb * 128) + rel * 4
                for comp, val in ((0, x), (1, y), (2, z), (3, w)):
                    ovi[...] = jnp.where(ovf, base + comp, VOXDUMPW)
                    ovv[...] = val
                    pltpu.async_copy(ovv, vox_out.at[ovi], sem).wait()

            return kc + jnp.sum(keep.astype(jnp.int32))

        emit_pass(jnp.int32(0), append_ovf)

    plsc.subcore_barrier()

    for c in range(30):
        @pl.when(t == c % NT)
        def _(c=c):
            nw = 2048 if c < 29 else 608
            w0 = c * 2048
            row0 = w0 // 3
            pltpu.sync_copy(cn_hbm.at[pl.ds(row0 * 8, 5504)],
                            rbuf.at[pl.ds(0, 5504)])

            def rp(j, _):
                wd = w0 + j * 16 + lane
                r = wd // 3
                src = (r - row0) * 8 + (wd - r * 3)
                v = plsc.load_gather(rbuf, [src])
                ostage[pl.ds(j * 16, 16)] = jnp.where(r < vn, v, -1)
                return 0

            lax.fori_loop(0, nw // 16, rp, 0)
            pltpu.sync_copy(ostage.at[pl.ds(0, nw)],
                            coor_out.at[pl.ds(w0, nw)])

    for c in range(20):
        @pl.when(t == c % NT)
        def _(c=c):
            nw = 1024 if c < 19 else 544
            w0 = c * 1024
            pltpu.sync_copy(cn_hbm.at[pl.ds(w0 * 8, 8192)], rbuf)

            def rp(j, _):
                wd = w0 + j * 16 + lane
                src = (wd - w0) * 8 + 3
                v = plsc.load_gather(rbuf, [src])
                ostage[pl.ds(j * 16, 16)] = jnp.where(wd < vn, v, 0)
                return 0

            lax.fori_loop(0, nw // 16, rp, 0)
            pltpu.sync_copy(ostage.at[pl.ds(0, nw)],
                            npv_out.at[pl.ds(w0, nw)])


def kernel(points):
    pts_flat = points.reshape(-1)
    vox, coor, npv, vnum = _vox_kernel(pts_flat)
    voxels = vox[: MAXV * MAXP * C].reshape(MAXV, MAXP, C)
    coors = coor[: MAXV * 3].reshape(MAXV, 3)
    return voxels, coors, npv[:MAXV], vnum[0]
